# Initial kernel scaffold; baseline (speedup 1.0000x reference)
import jax
import jax.numpy as jnp
from jax.experimental import pallas as pl


def kernel(x, batch, W1, b1, W2, b2):
    raise NotImplementedError("write your pallas kernel here")


# speedup vs baseline: 3.4664x; 3.4664x over previous
import functools
import jax
import jax.numpy as jnp
from jax.experimental import pallas as pl
from jax.experimental.pallas import tpu as pltpu

_N = 50000
_F = 256
_H = 4
_S = 512
_R = 2000
_NB = _N // _R
_NEG = -1e30


def _shift_down(v, d, fill):
    pad = jnp.full((d,) + v.shape[1:], fill, dtype=v.dtype)
    return jnp.concatenate([pad, v[:-d]], axis=0)


def _seg_cummax(s, ids):
    v = s
    cur = ids
    d = 1
    while d < _R:
        same = _shift_down(cur, d, jnp.int32(-1)) == cur
        vs = _shift_down(v, d, jnp.float32(_NEG))
        v = jnp.where(same, jnp.maximum(v, vs), v)
        d *= 2
    return v


def _onehot(ids):
    cols = jax.lax.broadcasted_iota(jnp.int32, (_R, _S), 1)
    return (cols == ids).astype(jnp.float32)


def _contract_rows(a, b):
    return jax.lax.dot_general(a, b, (((0,), (0,)), ((), ())),
                               preferred_element_type=jnp.float32)


def _stats_body(x_ref, ids_ref, w1_ref, b1_ref, w2_ref, b2_ref,
                s_ref, m_ref, sum_ref):
    w_eff = jnp.dot(w1_ref[...], w2_ref[...],
                    preferred_element_type=jnp.float32)
    b_eff = jnp.dot(b1_ref[...], w2_ref[...],
                    preferred_element_type=jnp.float32) + b2_ref[...]
    s = jnp.dot(x_ref[...], w_eff,
                preferred_element_type=jnp.float32) + b_eff
    s_ref[...] = s
    ids = ids_ref[0, 0, :].reshape(_R, 1)
    cmax = _seg_cummax(s, ids)
    nxt = jnp.concatenate([ids[1:], jnp.full((1, 1), -1, jnp.int32)], axis=0)
    last = (ids != nxt).astype(jnp.float32)
    oh = _onehot(ids)
    m_blk = _contract_rows(oh, cmax * last)
    present = _contract_rows(oh, last)
    m_blk = jnp.where(present > 0, m_blk, _NEG)
    m_gat = jnp.dot(oh, m_blk, preferred_element_type=jnp.float32)
    e = jnp.exp(s - m_gat)
    m_ref[0] = m_blk
    sum_ref[0] = _contract_rows(oh, e)


def _final_body(x_ref, ids_ref, s_ref, mp_ref, sp_ref, out_ref,
                m_sc, d_sc):
    i = pl.program_id(0)

    @pl.when(i == 0)
    def _():
        mp = mp_ref[...]
        sp = sp_ref[...]
        m_g = jnp.max(mp, axis=0)
        d_g = jnp.sum(sp * jnp.exp(mp - m_g[None]), axis=0)
        m_sc[...] = m_g
        d_sc[...] = d_g

    @pl.when(i > 0)
    def _():
        ids = ids_ref[0, 0, :].reshape(_R, 1)
        oh = _onehot(ids)
        m_n = jnp.dot(oh, m_sc[...], preferred_element_type=jnp.float32)
        d_n = jnp.dot(oh, d_sc[...], preferred_element_type=jnp.float32)
        e = jnp.exp(s_ref[...] - m_n)
        att = e / (d_n + 1e-16)
        w = jnp.sum(att, axis=1, keepdims=True)
        out_ref[...] = x_ref[...] * w


@jax.jit
def kernel(x, batch, W1, b1, W2, b2):
    ids3 = batch.astype(jnp.int32).reshape(_NB, 1, _R)
    b1r = b1.reshape(1, _F)
    b2r = b2.reshape(1, _H)

    s, m_part, sum_part = pl.pallas_call(
        _stats_body,
        grid=(_NB,),
        in_specs=[
            pl.BlockSpec((_R, _F), lambda i: (i, 0)),
            pl.BlockSpec((1, 1, _R), lambda i: (i, 0, 0)),
            pl.BlockSpec((_F, _F), lambda i: (0, 0)),
            pl.BlockSpec((1, _F), lambda i: (0, 0)),
            pl.BlockSpec((_F, _H), lambda i: (0, 0)),
            pl.BlockSpec((1, _H), lambda i: (0, 0)),
        ],
        out_specs=[
            pl.BlockSpec((_R, _H), lambda i: (i, 0)),
            pl.BlockSpec((1, _S, _H), lambda i: (i, 0, 0)),
            pl.BlockSpec((1, _S, _H), lambda i: (i, 0, 0)),
        ],
        out_shape=[
            jax.ShapeDtypeStruct((_N, _H), jnp.float32),
            jax.ShapeDtypeStruct((_NB, _S, _H), jnp.float32),
            jax.ShapeDtypeStruct((_NB, _S, _H), jnp.float32),
        ],
    )(x, ids3, W1, b1r, W2, b2r)

    out = pl.pallas_call(
        _final_body,
        grid=(_NB + 1,),
        in_specs=[
            pl.BlockSpec((_R, _F), lambda i: (jnp.maximum(i - 1, 0), 0)),
            pl.BlockSpec((1, 1, _R), lambda i: (jnp.maximum(i - 1, 0), 0, 0)),
            pl.BlockSpec((_R, _H), lambda i: (jnp.maximum(i - 1, 0), 0)),
            pl.BlockSpec((_NB, _S, _H), lambda i: (0, 0, 0)),
            pl.BlockSpec((_NB, _S, _H), lambda i: (0, 0, 0)),
        ],
        out_specs=pl.BlockSpec((_R, _F), lambda i: (jnp.maximum(i - 1, 0), 0)),
        out_shape=jax.ShapeDtypeStruct((_N, _F), jnp.float32),
        scratch_shapes=[
            pltpu.VMEM((_S, _H), jnp.float32),
            pltpu.VMEM((_S, _H), jnp.float32),
        ],
    )(x, ids3, s, m_part, sum_part)
    return out


# speedup vs baseline: 7.8507x; 2.2648x over previous
import jax
import jax.numpy as jnp
from jax.experimental import pallas as pl
from jax.experimental.pallas import tpu as pltpu

_N = 50000
_F = 256
_H = 4
_S = 512
_SA = 32
_SB = 16
_R = 2048
_NB = 25
_NPAD = _R * _NB
_NEG = -1e30


def _shift_right(v, d, fill):
    pad = jnp.full(v.shape[:-1] + (d,), fill, v.dtype)
    return jnp.concatenate([pad, v[..., :-d]], axis=-1)


def _seg_cummax(v, ids):
    cur = ids
    d = 1
    while d < _R:
        same = _shift_right(cur, d, jnp.int32(-1)) == cur
        vs = _shift_right(v, d, jnp.float32(_NEG))
        v = jnp.where(same, jnp.maximum(v, vs), v)
        d *= 2
    return v


def _dot(a, b, dims):
    return jax.lax.dot_general(a, b, (dims, ((), ())),
                               preferred_element_type=jnp.float32)


def _gather_T(stat, oa, ob):
    rows = []
    for h in range(_H):
        u = _dot(stat[h * _SA:(h + 1) * _SA], oa, ((0,), (0,)))
        rows.append(jnp.sum(u * ob, axis=0, keepdims=True))
    return jnp.concatenate(rows, axis=0)


def _scatter_T(vals, oa, ob):
    lhs = jnp.concatenate([oa * vals[h:h + 1] for h in range(_H)], axis=0)
    return _dot(lhs, ob, ((1,), (1,)))


def _masks(ids):
    oa = (jax.lax.broadcasted_iota(jnp.int32, (_SA, _R), 0)
          == (ids >> 4)).astype(jnp.float32)
    ob = (jax.lax.broadcasted_iota(jnp.int32, (_SB, _R), 0)
          == (ids & 15)).astype(jnp.float32)
    return oa, ob


def _body(x_ref, ids_ref, w1_ref, b1_ref, w2_ref, b2c_ref, out_ref,
          s_sc, m_sc, d_sc):
    i = pl.program_id(0)
    ids = ids_ref[...].reshape(1, _R)
    valid = ids < _S
    oa, ob = _masks(ids)

    @pl.when(i < _NB)
    def _phase_a():
        w_eff = jnp.dot(w1_ref[...], w2_ref[...],
                        preferred_element_type=jnp.float32)
        beff = _dot(w2_ref[...], b1_ref[...], ((0,), (1,)))
        s_t = _dot(w_eff, x_ref[...], ((0,), (1,))) + beff + b2c_ref[...]
        s_sc[i] = s_t
        v = jnp.where(valid, s_t, _NEG)
        cmax = _seg_cummax(v, ids)
        nxt = jnp.concatenate(
            [ids[:, 1:], jnp.full((1, 1), -1, jnp.int32)], axis=1)
        last = (ids != nxt).astype(jnp.float32)
        m_blk = _scatter_T(cmax * last, oa, ob)
        pres = _dot(oa * last, ob, ((1,), (1,)))
        pres = jnp.concatenate([pres] * _H, axis=0)
        m_blk = jnp.where(pres > 0, m_blk, _NEG)
        mg = _gather_T(m_blk, oa, ob)
        e = jnp.where(valid, jnp.exp(s_t - mg), 0.0)
        sum_blk = _scatter_T(e, oa, ob)

        @pl.when(i == 0)
        def _():
            m_sc[...] = jnp.full((_H * _SA, _SB), _NEG, jnp.float32)
            d_sc[...] = jnp.zeros((_H * _SA, _SB), jnp.float32)

        pm = m_sc[...]
        pd = d_sc[...]
        mn = jnp.maximum(pm, m_blk)
        d_sc[...] = pd * jnp.exp(pm - mn) + sum_blk * jnp.exp(m_blk - mn)
        m_sc[...] = mn

    @pl.when(i >= _NB)
    def _phase_b():
        s_t = s_sc[i - _NB]
        mg = _gather_T(m_sc[...], oa, ob)
        dg = _gather_T(d_sc[...], oa, ob)
        e = jnp.where(valid, jnp.exp(s_t - mg), 0.0)
        att = e / (dg + 1e-16)
        w = jnp.sum(att, axis=0, keepdims=True)
        out_ref[...] = x_ref[...] * w.reshape(_R, 1)


@jax.jit
def kernel(x, batch, W1, b1, W2, b2):
    ids3 = jnp.pad(batch.astype(jnp.int32), (0, _NPAD - _N),
                   constant_values=_S).reshape(_NB, 1, _R)
    b1r = b1.reshape(1, _F)
    b2c = b2.reshape(_H, 1)

    out = pl.pallas_call(
        _body,
        grid=(2 * _NB,),
        in_specs=[
            pl.BlockSpec((_R, _F),
                         lambda i: (jnp.where(i < _NB, i, i - _NB), 0)),
            pl.BlockSpec((1, 1, _R),
                         lambda i: (jnp.where(i < _NB, i, i - _NB), 0, 0)),
            pl.BlockSpec((_F, _F), lambda i: (0, 0)),
            pl.BlockSpec((1, _F), lambda i: (0, 0)),
            pl.BlockSpec((_F, _H), lambda i: (0, 0)),
            pl.BlockSpec((_H, 1), lambda i: (0, 0)),
        ],
        out_specs=pl.BlockSpec((_R, _F),
                               lambda i: (jnp.maximum(i - _NB, 0), 0)),
        out_shape=jax.ShapeDtypeStruct((_N, _F), jnp.float32),
        scratch_shapes=[
            pltpu.VMEM((_NB, _H, _R), jnp.float32),
            pltpu.VMEM((_H * _SA, _SB), jnp.float32),
            pltpu.VMEM((_H * _SA, _SB), jnp.float32),
        ],
    )(x, ids3, W1, b1r, W2, b2c)
    return out


# speedup vs baseline: 8.9807x; 1.1439x over previous
import jax
import jax.numpy as jnp
from jax.experimental import pallas as pl
from jax.experimental.pallas import tpu as pltpu

_N = 50000
_F = 256
_H = 4
_S = 512
_SA = 32
_SB = 16
_R = 2048
_NB = 25
_NPAD = _R * _NB
_NEG = -1e30


def _shift_right(v, d, fill):
    pad = jnp.full(v.shape[:-1] + (d,), fill, v.dtype)
    return jnp.concatenate([pad, v[..., :-d]], axis=-1)


def _seg_cummax(v, ids):
    cur = ids
    d = 1
    while d < _R:
        same = _shift_right(cur, d, jnp.int32(-1)) == cur
        vs = _shift_right(v, d, jnp.float32(_NEG))
        v = jnp.where(same, jnp.maximum(v, vs), v)
        d *= 2
    return v


def _dot(a, b, dims):
    return jax.lax.dot_general(a, b, (dims, ((), ())),
                               preferred_element_type=jnp.float32)


def _gather_T(stat, oa, ob):
    rows = []
    for h in range(_H):
        u = _dot(stat[h * _SA:(h + 1) * _SA], oa, ((0,), (0,)))
        rows.append(jnp.sum(u * ob, axis=0, keepdims=True))
    return jnp.concatenate(rows, axis=0)


def _scatter_T(vals, oa, ob):
    lhs = jnp.concatenate([oa * vals[h:h + 1] for h in range(_H)], axis=0)
    return _dot(lhs, ob, ((1,), (1,)))


def _masks(ids):
    oa = (jax.lax.broadcasted_iota(jnp.int32, (_SA, _R), 0)
          == (ids >> 4)).astype(jnp.float32)
    ob = (jax.lax.broadcasted_iota(jnp.int32, (_SB, _R), 0)
          == (ids & 15)).astype(jnp.float32)
    return oa, ob


def _body(x_ref, ids_ref, w1_ref, b1_ref, w2_ref, b2c_ref, out_ref,
          s_sc, m_sc, d_sc, xb_sc):
    i = pl.program_id(0)
    ids = ids_ref[...].reshape(1, _R)
    valid = ids < _S
    oa, ob = _masks(ids)

    @pl.when(i < _NB)
    def _phase_a():
        w_eff = jnp.dot(w1_ref[...], w2_ref[...],
                        preferred_element_type=jnp.float32)
        beff = _dot(w2_ref[...], b1_ref[...], ((0,), (1,)))
        xb = x_ref[...]
        s_t = _dot(w_eff, xb, ((0,), (1,))) + beff + b2c_ref[...]
        s_sc[i] = s_t
        xb_sc[i] = xb.astype(jnp.bfloat16)
        v = jnp.where(valid, s_t, _NEG)
        cmax = _seg_cummax(v, ids)
        nxt = jnp.concatenate(
            [ids[:, 1:], jnp.full((1, 1), -1, jnp.int32)], axis=1)
        last = (ids != nxt).astype(jnp.float32)
        m_blk = _scatter_T(cmax * last, oa, ob)
        pres = _dot(oa * last, ob, ((1,), (1,)))
        pres = jnp.concatenate([pres] * _H, axis=0)
        m_blk = jnp.where(pres > 0, m_blk, _NEG)
        mg = _gather_T(m_blk, oa, ob)
        e = jnp.where(valid, jnp.exp(s_t - mg), 0.0)
        sum_blk = _scatter_T(e, oa, ob)

        @pl.when(i == 0)
        def _():
            m_sc[...] = jnp.full((_H * _SA, _SB), _NEG, jnp.float32)
            d_sc[...] = jnp.zeros((_H * _SA, _SB), jnp.float32)

        pm = m_sc[...]
        pd = d_sc[...]
        mn = jnp.maximum(pm, m_blk)
        d_sc[...] = pd * jnp.exp(pm - mn) + sum_blk * jnp.exp(m_blk - mn)
        m_sc[...] = mn

    @pl.when(i >= _NB)
    def _phase_b():
        s_t = s_sc[i - _NB]
        mg = _gather_T(m_sc[...], oa, ob)
        dg = _gather_T(d_sc[...], oa, ob)
        e = jnp.where(valid, jnp.exp(s_t - mg), 0.0)
        att = e / (dg + 1e-16)
        w = jnp.sum(att, axis=0, keepdims=True)
        xb = xb_sc[i - _NB].astype(jnp.float32)
        out_ref[...] = xb * w.reshape(_R, 1)


@jax.jit
def kernel(x, batch, W1, b1, W2, b2):
    ids3 = jnp.pad(batch.astype(jnp.int32), (0, _NPAD - _N),
                   constant_values=_S).reshape(_NB, 1, _R)
    b1r = b1.reshape(1, _F)
    b2c = b2.reshape(_H, 1)

    out = pl.pallas_call(
        _body,
        grid=(2 * _NB,),
        in_specs=[
            pl.BlockSpec((_R, _F), lambda i: (jnp.minimum(i, _NB - 1), 0)),
            pl.BlockSpec((1, 1, _R),
                         lambda i: (jnp.where(i < _NB, i, i - _NB), 0, 0)),
            pl.BlockSpec((_F, _F), lambda i: (0, 0)),
            pl.BlockSpec((1, _F), lambda i: (0, 0)),
            pl.BlockSpec((_F, _H), lambda i: (0, 0)),
            pl.BlockSpec((_H, 1), lambda i: (0, 0)),
        ],
        out_specs=pl.BlockSpec((_R, _F),
                               lambda i: (jnp.maximum(i - _NB, 0), 0)),
        out_shape=jax.ShapeDtypeStruct((_N, _F), jnp.float32),
        scratch_shapes=[
            pltpu.VMEM((_NB, _H, _R), jnp.float32),
            pltpu.VMEM((_H * _SA, _SB), jnp.float32),
            pltpu.VMEM((_H * _SA, _SB), jnp.float32),
            pltpu.VMEM((_NB, _R, _F), jnp.bfloat16),
        ],
    )(x, ids3, W1, b1r, W2, b2c)
    return out
